# initial kernel scaffold (unmeasured)
import jax
import jax.numpy as jnp
from jax import lax
from jax.experimental import pallas as pl
from jax.experimental.pallas import tpu as pltpu


def kernel(
    x,
):
    def body(*refs):
        pass

    out_shape = jax.ShapeDtypeStruct(..., jnp.float32)
    return pl.pallas_call(body, out_shape=out_shape)(...)


# baseline (device time: 25740 ns/iter reference)
import jax
import jax.numpy as jnp
from jax import lax
from jax.experimental import pallas as pl
from jax.experimental.pallas import tpu as pltpu

N_DEV = 32
STAGES = 5


def kernel(x):
    _, m, n = x.shape

    def body(x_ref, out_ref, acc_ref, comm_ref, send_ref, send_sems, recv_sems):
        my = lax.axis_index("i")

        barrier_sem = pltpu.get_barrier_semaphore()
        for s in range(STAGES):
            peer = jnp.bitwise_xor(my, 1 << s)
            pl.semaphore_signal(
                barrier_sem,
                inc=1,
                device_id=(peer,),
                device_id_type=pl.DeviceIdType.MESH,
            )
        pl.semaphore_wait(barrier_sem, STAGES)

        acc_ref[...] = x_ref[0, :, :]

        for s in range(STAGES):
            peer = jnp.bitwise_xor(my, 1 << s)
            send_ref[s, :, :] = acc_ref[...].astype(jnp.bfloat16)
            rdma = pltpu.make_async_remote_copy(
                src_ref=send_ref.at[s],
                dst_ref=comm_ref.at[s],
                send_sem=send_sems.at[s],
                recv_sem=recv_sems.at[s],
                device_id=(peer,),
                device_id_type=pl.DeviceIdType.MESH,
            )
            rdma.start()
            rdma.wait()
            acc_ref[...] += comm_ref[s, :, :].astype(jnp.float32)

        out_ref[...] = acc_ref[...]

    return pl.pallas_call(
        body,
        out_shape=jax.ShapeDtypeStruct((m, n), jnp.float32),
        in_specs=[pl.BlockSpec(memory_space=pltpu.VMEM)],
        out_specs=pl.BlockSpec(memory_space=pltpu.VMEM),
        scratch_shapes=[
            pltpu.VMEM((m, n), jnp.float32),
            pltpu.VMEM((STAGES, m, n), jnp.bfloat16),
            pltpu.VMEM((STAGES, m, n), jnp.bfloat16),
            pltpu.SemaphoreType.DMA((STAGES,)),
            pltpu.SemaphoreType.DMA((STAGES,)),
        ],
        compiler_params=pltpu.CompilerParams(collective_id=0),
    )(x)


# device time: 25695 ns/iter; 1.0018x vs baseline; 1.0018x over previous
import jax
import jax.numpy as jnp
from jax import lax
from jax.experimental import pallas as pl
from jax.experimental.pallas import tpu as pltpu

N_DEV = 32
STAGES = 5


def kernel(x):
    _, m, n = x.shape

    def body(x_ref, out_ref, acc_ref, comm_ref, send_sems, recv_sems):
        my = lax.axis_index("i")

        barrier_sem = pltpu.get_barrier_semaphore()
        for s in range(STAGES):
            peer = jnp.bitwise_xor(my, 1 << s)
            pl.semaphore_signal(
                barrier_sem,
                inc=1,
                device_id=(peer,),
                device_id_type=pl.DeviceIdType.MESH,
            )
        pl.semaphore_wait(barrier_sem, STAGES)

        acc_ref[...] = x_ref[0, :, :].astype(jnp.bfloat16)

        for s in range(STAGES):
            peer = jnp.bitwise_xor(my, 1 << s)
            rdma = pltpu.make_async_remote_copy(
                src_ref=acc_ref,
                dst_ref=comm_ref.at[s],
                send_sem=send_sems.at[s],
                recv_sem=recv_sems.at[s],
                device_id=(peer,),
                device_id_type=pl.DeviceIdType.MESH,
            )
            rdma.start()
            rdma.wait()
            acc_ref[...] += comm_ref[s, :, :]

        out_ref[...] = acc_ref[...].astype(jnp.float32)

    return pl.pallas_call(
        body,
        out_shape=jax.ShapeDtypeStruct((m, n), jnp.float32),
        in_specs=[pl.BlockSpec(memory_space=pltpu.VMEM)],
        out_specs=pl.BlockSpec(memory_space=pltpu.VMEM),
        scratch_shapes=[
            pltpu.VMEM((m, n), jnp.bfloat16),
            pltpu.VMEM((STAGES, m, n), jnp.bfloat16),
            pltpu.SemaphoreType.DMA((STAGES,)),
            pltpu.SemaphoreType.DMA((STAGES,)),
        ],
        compiler_params=pltpu.CompilerParams(collective_id=0),
    )(x)


# device time: 22271 ns/iter; 1.1558x vs baseline; 1.1537x over previous
import jax
import jax.numpy as jnp
from jax import lax
from jax.experimental import pallas as pl
from jax.experimental.pallas import tpu as pltpu

N_DEV = 32
MASKS = (1, 3, 4, 8, 16)
STAGES = len(MASKS)
CHUNKS = 4


def kernel(x):
    _, m, n = x.shape
    rows = m // CHUNKS

    def body(x_ref, out_ref, acc_ref, comm_ref, send_sems, recv_sems):
        my = lax.axis_index("i")
        peers = [jnp.bitwise_xor(my, mk) for mk in MASKS]

        barrier_sem = pltpu.get_barrier_semaphore()
        for s in range(STAGES):
            pl.semaphore_signal(
                barrier_sem,
                inc=1,
                device_id=(peers[s],),
                device_id_type=pl.DeviceIdType.MESH,
            )
        pl.semaphore_wait(barrier_sem, STAGES)

        for c in range(CHUNKS):
            acc_ref[c, :, :] = x_ref[0, pl.ds(c * rows, rows), :].astype(
                jnp.bfloat16
            )

        def mk_rdma(s, c):
            return pltpu.make_async_remote_copy(
                src_ref=acc_ref.at[c],
                dst_ref=comm_ref.at[s, c],
                send_sem=send_sems.at[s, c],
                recv_sem=recv_sems.at[s, c],
                device_id=(peers[s],),
                device_id_type=pl.DeviceIdType.MESH,
            )

        rdmas = {}
        for c in range(CHUNKS):
            rdmas[(0, c)] = mk_rdma(0, c)
            rdmas[(0, c)].start()
        for s in range(STAGES):
            for c in range(CHUNKS):
                rdmas.pop((s, c)).wait()
                acc_ref[c, :, :] += comm_ref[s, c, :, :]
                if s + 1 < STAGES:
                    rdmas[(s + 1, c)] = mk_rdma(s + 1, c)
                    rdmas[(s + 1, c)].start()

        for c in range(CHUNKS):
            out_ref[pl.ds(c * rows, rows), :] = acc_ref[c, :, :].astype(
                jnp.float32
            )

    return pl.pallas_call(
        body,
        out_shape=jax.ShapeDtypeStruct((m, n), jnp.float32),
        in_specs=[pl.BlockSpec(memory_space=pltpu.VMEM)],
        out_specs=pl.BlockSpec(memory_space=pltpu.VMEM),
        scratch_shapes=[
            pltpu.VMEM((CHUNKS, rows, n), jnp.bfloat16),
            pltpu.VMEM((STAGES, CHUNKS, rows, n), jnp.bfloat16),
            pltpu.SemaphoreType.DMA((STAGES, CHUNKS)),
            pltpu.SemaphoreType.DMA((STAGES, CHUNKS)),
        ],
        compiler_params=pltpu.CompilerParams(collective_id=0),
    )(x)


# device time: 8992 ns/iter; 2.8625x vs baseline; 2.4768x over previous
import jax
import jax.numpy as jnp
from jax import lax
from jax.experimental import pallas as pl
from jax.experimental.pallas import tpu as pltpu

import os

N_DEV = 32
MASKS = tuple(
    int(v) for v in os.environ.get("KMASKS", "1,3,4,8,16").split(",") if v
)
STAGES = len(MASKS)
CHUNKS = int(os.environ.get("KCHUNKS", "4"))


def kernel(x):
    _, m, n = x.shape
    rows = m // CHUNKS

    def body(x_ref, out_ref, acc_ref, comm_ref, send_sems, recv_sems):
        my = lax.axis_index("i")
        peers = [jnp.bitwise_xor(my, mk) for mk in MASKS]

        barrier_sem = pltpu.get_barrier_semaphore()
        for s in range(STAGES):
            pl.semaphore_signal(
                barrier_sem,
                inc=1,
                device_id=(peers[s],),
                device_id_type=pl.DeviceIdType.MESH,
            )
        pl.semaphore_wait(barrier_sem, STAGES)

        for c in range(CHUNKS):
            acc_ref[c, :, :] = x_ref[0, pl.ds(c * rows, rows), :].astype(
                jnp.bfloat16
            )

        def mk_rdma(s, c):
            return pltpu.make_async_remote_copy(
                src_ref=acc_ref.at[c],
                dst_ref=comm_ref.at[s, c],
                send_sem=send_sems.at[s, c],
                recv_sem=recv_sems.at[s, c],
                device_id=(peers[s],),
                device_id_type=pl.DeviceIdType.MESH,
            )

        rdmas = {}
        for c in range(CHUNKS):
            rdmas[(0, c)] = mk_rdma(0, c)
            rdmas[(0, c)].start()
        for s in range(STAGES):
            for c in range(CHUNKS):
                rdmas.pop((s, c)).wait()
                acc_ref[c, :, :] += comm_ref[s, c, :, :]
                if s + 1 < STAGES:
                    rdmas[(s + 1, c)] = mk_rdma(s + 1, c)
                    rdmas[(s + 1, c)].start()

        for c in range(CHUNKS):
            out_ref[pl.ds(c * rows, rows), :] = acc_ref[c, :, :].astype(
                jnp.float32
            )

    return pl.pallas_call(
        body,
        out_shape=jax.ShapeDtypeStruct((m, n), jnp.float32),
        in_specs=[pl.BlockSpec(memory_space=pltpu.VMEM)],
        out_specs=pl.BlockSpec(memory_space=pltpu.VMEM),
        scratch_shapes=[
            pltpu.VMEM((CHUNKS, rows, n), jnp.bfloat16),
            pltpu.VMEM((STAGES, CHUNKS, rows, n), jnp.bfloat16),
            pltpu.SemaphoreType.DMA((STAGES, CHUNKS)),
            pltpu.SemaphoreType.DMA((STAGES, CHUNKS)),
        ],
        compiler_params=pltpu.CompilerParams(collective_id=0),
    )(x)
